# baseline (speedup 1.0000x reference)
import functools

import jax
import jax.numpy as jnp
from jax import lax
from jax.experimental import pallas as pl
from jax.experimental.pallas import tpu as pltpu
from jax.experimental.pallas import tpu_sc as plsc

B = 1
V = 10000
F = 20000
C = 128
NNZ = F * 3 * 16
K = 512
G = K // 128
PAD = 4 * K
NNZP = NNZ + PAD

f32 = jnp.float32
i32 = jnp.int32


def _elu(x):
    return jnp.where(x > 0, x, jnp.exp(jnp.minimum(x, 0.0)) - 1.0)


def _b16(x):
    return x.astype(jnp.bfloat16).astype(f32)


def _make_spmm(M, N):
    Mh = M // 2
    RPT = -(-(Mh // 16) // 8) * 8
    LAST = Mh - 15 * RPT
    ZR = 128
    zfull = LAST // ZR
    zrem_a, zrem_b = RPT - zfull * ZR, LAST - zfull * ZR
    mesh = plsc.VectorSubcoreMesh(core_axis_name="c", subcore_axis_name="s")

    @functools.partial(
        pl.kernel, mesh=mesh,
        compiler_params=pltpu.CompilerParams(use_tc_tiling_on_sc=False),
        out_type=jax.ShapeDtypeStruct((M, 32), f32),
        scratch_types=[
            pltpu.VMEM((272,), i32),
            pltpu.VMEM((K,), i32),
            pltpu.VMEM((K,), i32),
            pltpu.VMEM((K,), i32),
            pltpu.VMEM((K,), i32),
            pltpu.VMEM((G, 128), i32),
            pltpu.VMEM((G, 128), i32),
            pltpu.VMEM((K,), f32),
            pltpu.VMEM((K,), f32),
            pltpu.VMEM((K, 32), f32),
            pltpu.VMEM((K, 32), f32),
            pltpu.VMEM((G, 128), i32),
            pltpu.VMEM((G, 128), i32),
            pltpu.VMEM((ZR, 32), f32),
            pltpu.VMEM_SHARED((Mh + 8, 32), f32),
            pltpu.SemaphoreType.DMA,
            pltpu.SemaphoreType.DMA,
            pltpu.SemaphoreType.DMA,
            pltpu.SemaphoreType.DMA,
            pltpu.SemaphoreType.DMA,
            pltpu.SemaphoreType.DMA,
        ],
    )
    def spmm(rows_hbm, cols_hbm, vals_hbm, meta_hbm, table_hbm, out_hbm,
             meta_v, rows_a, rows_b, cols_a, cols_b, c2_a, c2_b,
             vals_a, vals_b, g_a, g_b, sidx_a, sidx_b, zero_v, acc,
             semT_a, semT_b, semG_a, semG_b, semS_a, semS_b):
        bufA = (rows_a, cols_a, c2_a, vals_a, g_a, sidx_a, semT_a, semG_a, semS_a)
        bufB = (rows_b, cols_b, c2_b, vals_b, g_b, sidx_b, semT_b, semG_b, semS_b)
        c = lax.axis_index("c")
        s = lax.axis_index("s")
        wid = c * 16 + s
        base_row = c * Mh

        z = jnp.zeros((16,), f32)

        def zfill(i, _):
            zero_v[i, pl.ds(0, 16)] = z
            zero_v[i, pl.ds(16, 16)] = z
            return 0

        lax.fori_loop(0, ZR, zfill, 0)
        zbase = pl.multiple_of(s * RPT, 8)
        for j in range(zfull):
            pltpu.sync_copy(zero_v, acc.at[pl.ds(zbase + j * ZR, ZR)])

        @pl.when(s < 15)
        def _():
            pltpu.sync_copy(zero_v.at[pl.ds(0, zrem_a)],
                            acc.at[pl.ds(zbase + zfull * ZR, zrem_a)])

        @pl.when(s == 15)
        def _():
            pltpu.sync_copy(zero_v.at[pl.ds(0, zrem_b)],
                            acc.at[pl.ds(zbase + zfull * ZR, zrem_b)])

        pltpu.sync_copy(meta_hbm, meta_v.at[pl.ds(0, 256)])
        plsc.subcore_barrier()

        mrow = meta_v[pl.ds(wid * 8, 16)]
        dma0 = mrow[0]
        npair = mrow[1]
        lo = mrow[2]
        hi = mrow[3]
        lane = lax.iota(i32, 16)

        def issue_triplets(j, buf):
            rows_x, cols_x, _, vals_x, _, _, semT, _, _ = buf
            cb = pl.multiple_of(dma0 + j * K, K)
            pltpu.async_copy(rows_hbm.at[pl.ds(cb, K)], rows_x, semT)
            pltpu.async_copy(cols_hbm.at[pl.ds(cb, K)], cols_x, semT)
            pltpu.async_copy(vals_hbm.at[pl.ds(cb, K)], vals_x, semT)

        def wait_triplets(buf):
            rows_x, cols_x, _, vals_x, _, _, semT, _, _ = buf
            pltpu.make_async_copy(rows_hbm.at[pl.ds(0, K)], rows_x, semT).wait()
            pltpu.make_async_copy(cols_hbm.at[pl.ds(0, K)], cols_x, semT).wait()
            pltpu.make_async_copy(vals_hbm.at[pl.ds(0, K)], vals_x, semT).wait()

        def fire_gathers(buf):
            _, cols_x, c2_x, _, g_x, _, _, semG, _ = buf
            for m in range(K // 16):
                c2_x[m // 8, pl.ds((m % 8) * 16, 16)] = cols_x[pl.ds(m * 16, 16)]
            return [pltpu.async_copy(table_hbm.at[c2_x.at[gi]],
                                     g_x.at[pl.ds(gi * 128, 128)], semG)
                    for gi in range(G)]

        def fire_scatter(buf):
            _, _, _, _, g_x, sidx_x, _, _, semS = buf
            return [pltpu.async_copy(g_x.at[pl.ds(gi * 128, 128)],
                                     acc.at[sidx_x.at[gi]], semS, add=True)
                    for gi in range(G)]

        def drain(handles):
            for h in handles:
                h.wait()

        def prep_and_scale(j, buf):
            rows_x, _, _, vals_x, g_x, sidx_x, _, _, _ = buf
            cb = dma0 + j * K
            for m in range(K // 16):
                gk = lane + (cb + m * 16)
                valid = (gk >= lo) & (gk < hi)
                r = rows_x[pl.ds(m * 16, 16)]
                li = jnp.where(valid, r - base_row, Mh)
                sidx_x[m // 8, pl.ds((m % 8) * 16, 16)] = li

            def mul16(t, _):
                k0 = t * 16
                vv16 = vals_x[pl.ds(k0, 16)]
                for u in range(16):
                    vv = vv16[u]
                    g_x[k0 + u, pl.ds(0, 16)] = g_x[k0 + u, pl.ds(0, 16)] * vv
                    g_x[k0 + u, pl.ds(16, 16)] = g_x[k0 + u, pl.ds(16, 16)] * vv
                return 0

            lax.fori_loop(0, K // 16, mul16, 0)

        issue_triplets(0, bufA)
        issue_triplets(1, bufB)

        def pair(jj, _):
            jA = jj * 2
            wait_triplets(bufA)
            hA = fire_gathers(bufA)
            wait_triplets(bufB)
            hB = fire_gathers(bufB)
            drain(hA)
            prep_and_scale(jA, bufA)
            sA = fire_scatter(bufA)
            drain(hB)
            prep_and_scale(jA + 1, bufB)
            sB = fire_scatter(bufB)
            issue_triplets(jA + 2, bufA)
            issue_triplets(jA + 3, bufB)
            drain(sA)
            drain(sB)
            return 0

        lax.fori_loop(0, npair, pair, 0)
        wait_triplets(bufA)
        wait_triplets(bufB)
        plsc.subcore_barrier()
        obase = pl.multiple_of(base_row + zbase, 8)

        @pl.when(s < 15)
        def _():
            pltpu.sync_copy(acc.at[pl.ds(zbase, RPT)],
                            out_hbm.at[pl.ds(obase, RPT)])

        @pl.when(s == 15)
        def _():
            pltpu.sync_copy(acc.at[pl.ds(zbase, LAST)],
                            out_hbm.at[pl.ds(obase, LAST)])

    return spmm


def _prep_triplets(rows, cols, vals, M):
    Mh = M // 2
    rows = rows.astype(i32)
    m0 = rows < Mh
    c = jnp.cumsum(m0.astype(i32))
    split = c[-1]
    i = jnp.arange(NNZ, dtype=i32)
    pos = jnp.where(m0, c - 1, split + i - c)
    rows_p = jnp.full((NNZP,), M, i32).at[pos].set(rows, unique_indices=True)
    cols_p = jnp.zeros((NNZP,), i32).at[pos].set(cols.astype(i32),
                                                 unique_indices=True)
    vals_p = jnp.zeros((NNZP,), f32).at[pos].set(vals, unique_indices=True)
    t = jnp.arange(17, dtype=i32)
    b0 = (split * t) // 16
    b1 = split + ((NNZ - split) * t) // 16
    lo = jnp.concatenate([b0[:-1], b1[:-1]])
    hi = jnp.concatenate([b0[1:], b1[1:]])
    dma = lo & (-K)
    npair = (hi - dma + (2 * K - 1)) // (2 * K)
    zero = jnp.zeros_like(dma)
    meta = jnp.stack([dma, npair, lo, hi, zero, zero, zero, zero],
                     axis=1).astype(i32).reshape(256)
    return rows_p, cols_p, vals_p, meta


def _make_stats(N, nin, BR):
    grid = N // BR

    def body(*refs):
        xs = refs[:nin]
        o_ref = refs[nin]
        i = pl.program_id(0)

        @pl.when(i == 0)
        def _():
            o_ref[...] = jnp.zeros_like(o_ref)

        parts = []
        for xr in xs:
            x = xr[...]
            parts.append(jnp.sum(x, axis=0, keepdims=True))
            parts.append(jnp.sum(x * x, axis=0, keepdims=True))
        parts.append(jnp.zeros((8 - 2 * nin, C), f32))
        o_ref[...] += jnp.concatenate(parts, axis=0)

    return pl.pallas_call(
        body,
        grid=(grid,),
        in_specs=[pl.BlockSpec((BR, C), lambda i: (i, 0)) for _ in range(nin)],
        out_specs=pl.BlockSpec((8, C), lambda i: (0, 0)),
        out_shape=jax.ShapeDtypeStruct((8, C), f32),
    )


def _make_mm(N, nstreams, has_res, BR):
    grid = N // BR
    nr = nstreams + (1 if has_res else 0)

    def body(*refs):
        xs = refs[:nstreams]
        k = nstreams
        r_ref = refs[k] if has_res else None
        k += 1 if has_res else 0
        ws = refs[k:k + nstreams]
        nm_ref = refs[k + nstreams]
        bias_ref = refs[k + nstreams + 1]
        o_ref, e_ref = refs[-2], refs[-1]
        acc = None
        for j, (xr, wr) in enumerate(zip(xs, ws)):
            xn = (xr[...] - nm_ref[3 * j, :]) * nm_ref[3 * j + 1, :] + nm_ref[3 * j + 2, :]
            d = jnp.dot(_b16(xn), _b16(wr[...]), preferred_element_type=f32)
            acc = d if acc is None else acc + d
        acc = acc + bias_ref[...]
        if has_res:
            acc = acc + r_ref[...]
        o_ref[...] = acc
        e_ref[...] = _elu(acc)

    in_specs = ([pl.BlockSpec((BR, C), lambda i: (i, 0)) for _ in range(nr)]
                + [pl.BlockSpec((C, C), lambda i: (0, 0)) for _ in range(nstreams)]
                + [pl.BlockSpec((8, C), lambda i: (0, 0)),
                   pl.BlockSpec((1, C), lambda i: (0, 0))])
    return pl.pallas_call(
        body,
        grid=(grid,),
        in_specs=in_specs,
        out_specs=[pl.BlockSpec((BR, C), lambda i: (i, 0))] * 2,
        out_shape=[jax.ShapeDtypeStruct((N, C), f32)] * 2,
    )


def _make_conv1(BR):
    grid = V // BR

    def body(x_ref, w_ref, b_ref, o_ref, e_ref):
        acc = jnp.dot(_b16(x_ref[...]), _b16(w_ref[...]),
                      preferred_element_type=f32)
        acc = acc + b_ref[...]
        o_ref[...] = acc
        e_ref[...] = _elu(acc)

    return pl.pallas_call(
        body,
        grid=(grid,),
        in_specs=[pl.BlockSpec((BR, 3), lambda i: (i, 0)),
                  pl.BlockSpec((3, C), lambda i: (0, 0)),
                  pl.BlockSpec((1, C), lambda i: (0, 0))],
        out_specs=[pl.BlockSpec((BR, C), lambda i: (i, 0))] * 2,
        out_shape=[jax.ShapeDtypeStruct((V, C), f32)] * 2,
    )


def _make_conv2(BR):
    grid = V // BR

    def body(x_ref, nm_ref, w_ref, b_ref, o_ref):
        x = x_ref[...]
        xn = (x - nm_ref[0, :]) * nm_ref[1, :] + nm_ref[2, :]
        acc = jnp.sum(_b16(xn) * _b16(w_ref[...]), axis=1, keepdims=True)
        acc = acc + b_ref[0, 0]
        o_ref[...] = _elu(acc)

    return pl.pallas_call(
        body,
        grid=(grid,),
        in_specs=[pl.BlockSpec((BR, C), lambda i: (i, 0)),
                  pl.BlockSpec((8, C), lambda i: (0, 0)),
                  pl.BlockSpec((1, C), lambda i: (0, 0)),
                  pl.BlockSpec((1, 1), lambda i: (0, 0))],
        out_specs=pl.BlockSpec((BR, 1), lambda i: (i, 0)),
        out_shape=jax.ShapeDtypeStruct((V, 1), f32),
    )


def _norm_rows(s, q, n, gamma, beta):
    mu = s / n
    var = q / n - mu * mu
    sc = gamma / jnp.sqrt(var + 1e-5)
    return mu, sc, beta


def _nm(*rows):
    pad = [jnp.zeros((C,), f32)] * (8 - len(rows))
    return jnp.stack(list(rows) + pad, axis=0)


def kernel(Di_rows, Di_cols, Di_vals, DiA_rows, DiA_cols, DiA_vals, mask, inputs, params):
    spmm_di = _make_spmm(4 * F, 4 * V)
    spmm_da = _make_spmm(4 * V, 4 * F)
    stats1_v = _make_stats(V, 1, 2000)
    stats2_v = _make_stats(V, 2, 2000)
    stats2_f = _make_stats(F, 2, 2000)
    mm2_f = _make_mm(F, 2, False, 2000)
    mm2res_v = _make_mm(V, 2, True, 2000)
    mm1_v = _make_mm(V, 1, False, 2000)
    mm1res_v = _make_mm(V, 1, True, 2000)
    conv1 = _make_conv1(1000)
    conv2 = _make_conv2(1000)

    di = _prep_triplets(Di_rows, Di_cols, Di_vals, 4 * F)
    da = _prep_triplets(DiA_rows, DiA_cols, DiA_vals, 4 * V)

    p1 = params['conv1']
    v, ev = conv1(inputs.reshape(V, 3), p1['W'], p1['b'].reshape(1, C))
    f = jnp.zeros((F, C), f32)

    for i in range(15):
        pr = params['rn%d' % i]
        if i % 2 == 0:
            p0, p1b = pr['fc0'], pr['fc1']
            sf = spmm_di(di[0], di[1], di[2], di[3], ev.reshape(4 * V, 32))
            sf = sf.reshape(F, C)
            st = stats2_f(f, sf)
            na = _norm_rows(st[0], st[1], F, p0['gamma'][:C], p0['beta'][:C])
            nb = _norm_rows(st[2], st[3], F, p0['gamma'][C:], p0['beta'][C:])
            x, ex = mm2_f(f, sf, p0['W'][:C], p0['W'][C:], _nm(*na, *nb),
                          p0['b'].reshape(1, C))
            sv = spmm_da(da[0], da[1], da[2], da[3], ex.reshape(4 * F, 32))
            sv = sv.reshape(V, C)
            st2 = stats2_v(v, sv)
            nc = _norm_rows(st2[0], st2[1], V, p1b['gamma'][:C], p1b['beta'][:C])
            nd = _norm_rows(st2[2], st2[3], V, p1b['gamma'][C:], p1b['beta'][C:])
            v, ev = mm2res_v(v, sv, v, p1b['W'][:C], p1b['W'][C:],
                             _nm(*nc, *nd), p1b['b'].reshape(1, C))
            f = x
        else:
            v_in = v
            e_cur = ev
            for p in (pr['fc0'], pr['fc1']):
                st = stats1_v(e_cur)
                na = _norm_rows(st[0], st[1], V, p['gamma'][:C], p['beta'][:C])
                bias = (p['b'] + jnp.dot(_b16(p['beta'][C:]), _b16(p['W'][C:]))
                        ).reshape(1, C)
                if p is pr['fc1']:
                    v, ev = mm1res_v(e_cur, v_in, p['W'][:C], _nm(*na), bias)
                else:
                    _, e_cur = mm1_v(e_cur, p['W'][:C], _nm(*na), bias)

    st = stats1_v(ev)
    p2 = params['conv2']
    n2 = _norm_rows(st[0], st[1], V, p2['gamma'], p2['beta'])
    out = conv2(ev, _nm(*n2), p2['W'].reshape(1, C), p2['b'].reshape(1, 1))
    return out.reshape(B, V, 1)


# speedup vs baseline: 4.7441x; 4.7441x over previous
import functools

import jax
import jax.numpy as jnp
from jax import lax
from jax.experimental import pallas as pl
from jax.experimental.pallas import tpu as pltpu
from jax.experimental.pallas import tpu_sc as plsc

B = 1
V = 10000
F = 20000
C = 128
NNZ = F * 3 * 16
K = 512
G = K // 128
PAD = 4 * K
NNZP = NNZ + PAD

f32 = jnp.float32
i32 = jnp.int32


def _elu(x):
    return jnp.where(x > 0, x, jnp.exp(jnp.minimum(x, 0.0)) - 1.0)


def _b16(x):
    return x.astype(jnp.bfloat16).astype(f32)


def _make_spmm(M, N, replicated=False):
    Mh = M if replicated else M // 2
    RPT = -(-(Mh // 16) // 8) * 8
    LAST = Mh - 15 * RPT
    ZR = 128
    zfull = LAST // ZR
    zrem_a, zrem_b = RPT - zfull * ZR, LAST - zfull * ZR
    out_shape = (2, M, 32) if replicated else (M, 32)
    mesh = plsc.VectorSubcoreMesh(core_axis_name="c", subcore_axis_name="s")

    @functools.partial(
        pl.kernel, mesh=mesh,
        compiler_params=pltpu.CompilerParams(use_tc_tiling_on_sc=False),
        out_type=jax.ShapeDtypeStruct(out_shape, f32),
        scratch_types=[
            pltpu.VMEM((272,), i32),
            pltpu.VMEM((K,), i32),
            pltpu.VMEM((K,), i32),
            pltpu.VMEM((K,), i32),
            pltpu.VMEM((K,), i32),
            pltpu.VMEM((G, 128), i32),
            pltpu.VMEM((G, 128), i32),
            pltpu.VMEM((K,), f32),
            pltpu.VMEM((K,), f32),
            pltpu.VMEM((K, 32), f32),
            pltpu.VMEM((K, 32), f32),
            pltpu.VMEM((G, 128), i32),
            pltpu.VMEM((G, 128), i32),
            pltpu.VMEM((ZR, 32), f32),
            pltpu.VMEM_SHARED((Mh + 8, 32), f32),
            pltpu.SemaphoreType.DMA,
            pltpu.SemaphoreType.DMA,
            pltpu.SemaphoreType.DMA,
            pltpu.SemaphoreType.DMA,
            pltpu.SemaphoreType.DMA,
            pltpu.SemaphoreType.DMA,
        ],
    )
    def spmm(rows_hbm, cols_hbm, vals_hbm, meta_hbm, table_hbm, out_hbm,
             meta_v, rows_a, rows_b, cols_a, cols_b, c2_a, c2_b,
             vals_a, vals_b, g_a, g_b, sidx_a, sidx_b, zero_v, acc,
             semT_a, semT_b, semG_a, semG_b, semS_a, semS_b):
        bufA = (rows_a, cols_a, c2_a, vals_a, g_a, sidx_a, semT_a, semG_a, semS_a)
        bufB = (rows_b, cols_b, c2_b, vals_b, g_b, sidx_b, semT_b, semG_b, semS_b)
        c = lax.axis_index("c")
        s = lax.axis_index("s")
        wid = c * 16 + s
        base_row = c * (0 if replicated else Mh)

        z = jnp.zeros((16,), f32)

        def zfill(i, _):
            zero_v[i, pl.ds(0, 16)] = z
            zero_v[i, pl.ds(16, 16)] = z
            return 0

        lax.fori_loop(0, ZR, zfill, 0)
        zbase = pl.multiple_of(s * RPT, 8)
        for j in range(zfull):
            pltpu.sync_copy(zero_v, acc.at[pl.ds(zbase + j * ZR, ZR)])

        @pl.when(s < 15)
        def _():
            pltpu.sync_copy(zero_v.at[pl.ds(0, zrem_a)],
                            acc.at[pl.ds(zbase + zfull * ZR, zrem_a)])

        @pl.when(s == 15)
        def _():
            pltpu.sync_copy(zero_v.at[pl.ds(0, zrem_b)],
                            acc.at[pl.ds(zbase + zfull * ZR, zrem_b)])

        pltpu.sync_copy(meta_hbm, meta_v.at[pl.ds(0, 256)])
        plsc.subcore_barrier()

        mrow = meta_v[pl.ds(wid * 8, 16)]
        dma0 = mrow[0]
        npair = mrow[1]
        lo = mrow[2]
        hi = mrow[3]
        lane = lax.iota(i32, 16)

        def issue_triplets(j, buf):
            rows_x, cols_x, _, vals_x, _, _, semT, _, _ = buf
            cb = pl.multiple_of(dma0 + j * K, K)
            pltpu.async_copy(rows_hbm.at[pl.ds(cb, K)], rows_x, semT)
            pltpu.async_copy(cols_hbm.at[pl.ds(cb, K)], cols_x, semT)
            pltpu.async_copy(vals_hbm.at[pl.ds(cb, K)], vals_x, semT)

        def wait_triplets(buf):
            rows_x, cols_x, _, vals_x, _, _, semT, _, _ = buf
            pltpu.make_async_copy(rows_hbm.at[pl.ds(0, K)], rows_x, semT).wait()
            pltpu.make_async_copy(cols_hbm.at[pl.ds(0, K)], cols_x, semT).wait()
            pltpu.make_async_copy(vals_hbm.at[pl.ds(0, K)], vals_x, semT).wait()

        def fire_gathers(buf):
            _, cols_x, c2_x, _, g_x, _, _, semG, _ = buf
            for m in range(K // 16):
                c2_x[m // 8, pl.ds((m % 8) * 16, 16)] = cols_x[pl.ds(m * 16, 16)]
            return [pltpu.async_copy(table_hbm.at[c2_x.at[gi]],
                                     g_x.at[pl.ds(gi * 128, 128)], semG)
                    for gi in range(G)]

        def fire_scatter(buf):
            _, _, _, _, g_x, sidx_x, _, _, semS = buf
            return [pltpu.async_copy(g_x.at[pl.ds(gi * 128, 128)],
                                     acc.at[sidx_x.at[gi]], semS, add=True)
                    for gi in range(G)]

        def drain(handles):
            for h in handles:
                h.wait()

        def prep_and_scale(j, buf):
            rows_x, _, _, vals_x, g_x, sidx_x, _, _, _ = buf
            cb = dma0 + j * K
            for m in range(K // 16):
                gk = lane + (cb + m * 16)
                valid = (gk >= lo) & (gk < hi)
                r = rows_x[pl.ds(m * 16, 16)]
                li = jnp.where(valid, r - base_row, Mh)
                sidx_x[m // 8, pl.ds((m % 8) * 16, 16)] = li

            def mul16(t, _):
                k0 = t * 16
                vv16 = vals_x[pl.ds(k0, 16)]
                for u in range(16):
                    vv = vv16[u]
                    g_x[k0 + u, pl.ds(0, 16)] = g_x[k0 + u, pl.ds(0, 16)] * vv
                    g_x[k0 + u, pl.ds(16, 16)] = g_x[k0 + u, pl.ds(16, 16)] * vv
                return 0

            lax.fori_loop(0, K // 16, mul16, 0)

        issue_triplets(0, bufA)
        issue_triplets(1, bufB)

        def pair(jj, _):
            jA = jj * 2
            wait_triplets(bufA)
            hA = fire_gathers(bufA)
            wait_triplets(bufB)
            hB = fire_gathers(bufB)
            drain(hA)
            prep_and_scale(jA, bufA)
            sA = fire_scatter(bufA)
            drain(hB)
            prep_and_scale(jA + 1, bufB)
            sB = fire_scatter(bufB)
            issue_triplets(jA + 2, bufA)
            issue_triplets(jA + 3, bufB)
            drain(sA)
            drain(sB)
            return 0

        lax.fori_loop(0, npair, pair, 0)
        wait_triplets(bufA)
        wait_triplets(bufB)
        plsc.subcore_barrier()
        if replicated:
            out_dst = out_hbm.at[c]
            obase = zbase
        else:
            out_dst = out_hbm
            obase = pl.multiple_of(base_row + zbase, 8)

        @pl.when(s < 15)
        def _():
            pltpu.sync_copy(acc.at[pl.ds(zbase, RPT)],
                            out_dst.at[pl.ds(obase, RPT)])

        @pl.when(s == 15)
        def _():
            pltpu.sync_copy(acc.at[pl.ds(zbase, LAST)],
                            out_dst.at[pl.ds(obase, LAST)])

    return spmm


def _meta_from_bounds(lo, hi):
    dma = lo & (-K)
    npair = (hi - dma + (2 * K - 1)) // (2 * K)
    zero = jnp.zeros_like(dma)
    return jnp.stack([dma, npair, lo, hi, zero, zero, zero, zero],
                     axis=1).astype(i32).reshape(256)


def _prep_triplets(rows, cols, vals, M):
    order = jnp.argsort(rows)
    rs = rows[order].astype(i32)
    cs = cols[order].astype(i32)
    vs = vals[order]
    Mh = M // 2
    split = jnp.searchsorted(rs, Mh, side='left').astype(i32)
    t = jnp.arange(17, dtype=i32)
    b0 = (split * t) // 16
    b1 = split + ((NNZ - split) * t) // 16
    lo = jnp.concatenate([b0[:-1], b1[:-1]])
    hi = jnp.concatenate([b0[1:], b1[1:]])
    rows_p = jnp.concatenate([rs, jnp.full((PAD,), M, i32)])
    cols_p = jnp.concatenate([cs, jnp.zeros((PAD,), i32)])
    vals_p = jnp.concatenate([vs, jnp.zeros((PAD,), f32)])
    return rows_p, cols_p, vals_p, _meta_from_bounds(lo, hi)


def _prep_triplets_replicated(rows, cols, vals, M):
    lo = jnp.arange(32, dtype=i32) * (NNZ // 32)
    hi = lo + (NNZ // 32)
    rows_p = jnp.concatenate([rows.astype(i32), jnp.full((PAD,), M, i32)])
    cols_p = jnp.concatenate([cols.astype(i32), jnp.zeros((PAD,), i32)])
    vals_p = jnp.concatenate([vals, jnp.zeros((PAD,), f32)])
    return rows_p, cols_p, vals_p, _meta_from_bounds(lo, hi)


def _make_stats(N, nin, BR):
    grid = N // BR

    def body(*refs):
        xs = refs[:nin]
        o_ref = refs[nin]
        i = pl.program_id(0)

        @pl.when(i == 0)
        def _():
            o_ref[...] = jnp.zeros_like(o_ref)

        parts = []
        for xr in xs:
            x = xr[...]
            parts.append(jnp.sum(x, axis=0, keepdims=True))
            parts.append(jnp.sum(x * x, axis=0, keepdims=True))
        parts.append(jnp.zeros((8 - 2 * nin, C), f32))
        o_ref[...] += jnp.concatenate(parts, axis=0)

    return pl.pallas_call(
        body,
        grid=(grid,),
        in_specs=[pl.BlockSpec((BR, C), lambda i: (i, 0)) for _ in range(nin)],
        out_specs=pl.BlockSpec((8, C), lambda i: (0, 0)),
        out_shape=jax.ShapeDtypeStruct((8, C), f32),
    )


def _make_stats2s(N, BR):
    grid = N // BR

    def body(x1_ref, y0_ref, y1_ref, o_ref, ys_ref):
        i = pl.program_id(0)

        @pl.when(i == 0)
        def _():
            o_ref[...] = jnp.zeros_like(o_ref)

        x1 = x1_ref[...]
        y = y0_ref[...] + y1_ref[...]
        ys_ref[...] = y
        o_ref[...] += jnp.concatenate(
            [jnp.sum(x1, axis=0, keepdims=True),
             jnp.sum(x1 * x1, axis=0, keepdims=True),
             jnp.sum(y, axis=0, keepdims=True),
             jnp.sum(y * y, axis=0, keepdims=True),
             jnp.zeros((4, C), f32)], axis=0)

    return pl.pallas_call(
        body,
        grid=(grid,),
        in_specs=[pl.BlockSpec((BR, C), lambda i: (i, 0)) for _ in range(3)],
        out_specs=[pl.BlockSpec((8, C), lambda i: (0, 0)),
                   pl.BlockSpec((BR, C), lambda i: (i, 0))],
        out_shape=[jax.ShapeDtypeStruct((8, C), f32),
                   jax.ShapeDtypeStruct((N, C), f32)],
    )


def _make_mm(N, nstreams, has_res, BR):
    grid = N // BR
    nr = nstreams + (1 if has_res else 0)

    def body(*refs):
        xs = refs[:nstreams]
        k = nstreams
        r_ref = refs[k] if has_res else None
        k += 1 if has_res else 0
        ws = refs[k:k + nstreams]
        nm_ref = refs[k + nstreams]
        bias_ref = refs[k + nstreams + 1]
        o_ref, e_ref = refs[-2], refs[-1]
        acc = None
        for j, (xr, wr) in enumerate(zip(xs, ws)):
            xn = (xr[...] - nm_ref[3 * j, :]) * nm_ref[3 * j + 1, :] + nm_ref[3 * j + 2, :]
            d = jnp.dot(_b16(xn), _b16(wr[...]), preferred_element_type=f32)
            acc = d if acc is None else acc + d
        acc = acc + bias_ref[...]
        if has_res:
            acc = acc + r_ref[...]
        o_ref[...] = acc
        e_ref[...] = _elu(acc)

    in_specs = ([pl.BlockSpec((BR, C), lambda i: (i, 0)) for _ in range(nr)]
                + [pl.BlockSpec((C, C), lambda i: (0, 0)) for _ in range(nstreams)]
                + [pl.BlockSpec((8, C), lambda i: (0, 0)),
                   pl.BlockSpec((1, C), lambda i: (0, 0))])
    return pl.pallas_call(
        body,
        grid=(grid,),
        in_specs=in_specs,
        out_specs=[pl.BlockSpec((BR, C), lambda i: (i, 0))] * 2,
        out_shape=[jax.ShapeDtypeStruct((N, C), f32)] * 2,
    )


def _make_conv1(BR):
    grid = V // BR

    def body(x_ref, w_ref, b_ref, o_ref, e_ref):
        acc = jnp.dot(_b16(x_ref[...]), _b16(w_ref[...]),
                      preferred_element_type=f32)
        acc = acc + b_ref[...]
        o_ref[...] = acc
        e_ref[...] = _elu(acc)

    return pl.pallas_call(
        body,
        grid=(grid,),
        in_specs=[pl.BlockSpec((BR, 3), lambda i: (i, 0)),
                  pl.BlockSpec((3, C), lambda i: (0, 0)),
                  pl.BlockSpec((1, C), lambda i: (0, 0))],
        out_specs=[pl.BlockSpec((BR, C), lambda i: (i, 0))] * 2,
        out_shape=[jax.ShapeDtypeStruct((V, C), f32)] * 2,
    )


def _make_conv2(BR):
    grid = V // BR

    def body(x_ref, nm_ref, w_ref, b_ref, o_ref):
        x = x_ref[...]
        xn = (x - nm_ref[0, :]) * nm_ref[1, :] + nm_ref[2, :]
        acc = jnp.sum(_b16(xn) * _b16(w_ref[...]), axis=1, keepdims=True)
        acc = acc + b_ref[0, 0]
        o_ref[...] = _elu(acc)

    return pl.pallas_call(
        body,
        grid=(grid,),
        in_specs=[pl.BlockSpec((BR, C), lambda i: (i, 0)),
                  pl.BlockSpec((8, C), lambda i: (0, 0)),
                  pl.BlockSpec((1, C), lambda i: (0, 0)),
                  pl.BlockSpec((1, 1), lambda i: (0, 0))],
        out_specs=pl.BlockSpec((BR, 1), lambda i: (i, 0)),
        out_shape=jax.ShapeDtypeStruct((V, 1), f32),
    )


def _norm_rows(s, q, n, gamma, beta):
    mu = s / n
    var = q / n - mu * mu
    sc = gamma / jnp.sqrt(var + 1e-5)
    return mu, sc, beta


def _nm(*rows):
    pad = [jnp.zeros((C,), f32)] * (8 - len(rows))
    return jnp.stack(list(rows) + pad, axis=0)


def kernel(Di_rows, Di_cols, Di_vals, DiA_rows, DiA_cols, DiA_vals, mask, inputs, params):
    spmm_di = _make_spmm(4 * F, 4 * V)
    spmm_da = _make_spmm(4 * V, 4 * F, replicated=True)
    stats1_v = _make_stats(V, 1, 2000)
    stats2s_v = _make_stats2s(V, 2000)
    stats2_f = _make_stats(F, 2, 2000)
    mm2_f = _make_mm(F, 2, False, 2000)
    mm2res_v = _make_mm(V, 2, True, 2000)
    mm1_v = _make_mm(V, 1, False, 2000)
    mm1res_v = _make_mm(V, 1, True, 2000)
    conv1 = _make_conv1(1000)
    conv2 = _make_conv2(1000)

    di = _prep_triplets(Di_rows, Di_cols, Di_vals, 4 * F)
    da = _prep_triplets_replicated(DiA_rows, DiA_cols, DiA_vals, 4 * V)

    p1 = params['conv1']
    v, ev = conv1(inputs.reshape(V, 3), p1['W'], p1['b'].reshape(1, C))
    f = jnp.zeros((F, C), f32)

    for i in range(15):
        pr = params['rn%d' % i]
        if i % 2 == 0:
            p0, p1b = pr['fc0'], pr['fc1']
            sf = spmm_di(di[0], di[1], di[2], di[3], ev.reshape(4 * V, 32))
            sf = sf.reshape(F, C)
            st = stats2_f(f, sf)
            na = _norm_rows(st[0], st[1], F, p0['gamma'][:C], p0['beta'][:C])
            nb = _norm_rows(st[2], st[3], F, p0['gamma'][C:], p0['beta'][C:])
            x, ex = mm2_f(f, sf, p0['W'][:C], p0['W'][C:], _nm(*na, *nb),
                          p0['b'].reshape(1, C))
            sv2 = spmm_da(da[0], da[1], da[2], da[3], ex.reshape(4 * F, 32))
            st2, sv = stats2s_v(v, sv2[0].reshape(V, C), sv2[1].reshape(V, C))
            nc = _norm_rows(st2[0], st2[1], V, p1b['gamma'][:C], p1b['beta'][:C])
            nd = _norm_rows(st2[2], st2[3], V, p1b['gamma'][C:], p1b['beta'][C:])
            v, ev = mm2res_v(v, sv, v, p1b['W'][:C], p1b['W'][C:],
                             _nm(*nc, *nd), p1b['b'].reshape(1, C))
            f = x
        else:
            v_in = v
            e_cur = ev
            for p in (pr['fc0'], pr['fc1']):
                st = stats1_v(e_cur)
                na = _norm_rows(st[0], st[1], V, p['gamma'][:C], p['beta'][:C])
                bias = (p['b'] + jnp.dot(_b16(p['beta'][C:]), _b16(p['W'][C:]))
                        ).reshape(1, C)
                if p is pr['fc1']:
                    v, ev = mm1res_v(e_cur, v_in, p['W'][:C], _nm(*na), bias)
                else:
                    _, e_cur = mm1_v(e_cur, p['W'][:C], _nm(*na), bias)

    st = stats1_v(ev)
    p2 = params['conv2']
    n2 = _norm_rows(st[0], st[1], V, p2['gamma'], p2['beta'])
    out = conv2(ev, _nm(*n2), p2['W'].reshape(1, C), p2['b'].reshape(1, 1))
    return out.reshape(B, V, 1)
